# baseline (speedup 1.0000x reference)
import jax
import jax.numpy as jnp
from jax.experimental import pallas as pl
from jax.experimental.pallas import tpu as pltpu

_NUM_CLASSES = 80
_ATTRS = 5 + _NUM_CLASSES
_INPUT_SIZE = 608.0


def _decode_body(W, H, n_anch):
    invW = 1.0 / W
    invH = 1.0 / H
    stride_w = _INPUT_SIZE / W
    stride_h = _INPUT_SIZE / H
    HW = H * W

    def body(x_ref, anch_ref, o_ref):
        s = jax.lax.broadcasted_iota(jnp.int32, (1, HW), 1)
        gx = (s % W).astype(jnp.float32)
        gy = (s // W).astype(jnp.float32)
        for a in range(n_anch):
            v = x_ref[:, :, 0, a, :]
            rt = v.reshape(HW, _ATTRS).T
            ex = jnp.exp(rt)
            sig = ex / (1.0 + ex)
            aw = anch_ref[0, 6 + a] * (invW / stride_w)
            ah = anch_ref[1, 6 + a] * (invH / stride_h)
            dec = jnp.concatenate([
                (sig[0:1] + gx) * invW,
                (sig[1:2] + gy) * invH,
                ex[2:3] * aw,
                ex[3:4] * ah,
                sig[4:],
            ], axis=0)
            o_ref[:, 0, a, :] = dec

    return body


def kernel(inputs_1, anchors):
    B, C, H, W = inputs_1.shape
    n_anch = 3
    HW = H * W

    xt = jnp.transpose(inputs_1, (2, 3, 0, 1)).reshape(H, W, B, n_anch, _ATTRS)
    anch_t = anchors.T

    out = pl.pallas_call(
        _decode_body(W, H, n_anch),
        grid=(B,),
        in_specs=[
            pl.BlockSpec((H, W, 1, n_anch, _ATTRS), lambda b: (0, 0, b, 0, 0)),
            pl.BlockSpec(memory_space=pltpu.SMEM),
        ],
        out_specs=pl.BlockSpec((_ATTRS, 1, n_anch, HW), lambda b: (0, b, 0, 0)),
        out_shape=jax.ShapeDtypeStruct((_ATTRS, B, n_anch, HW), jnp.float32),
    )(xt, anch_t)
    return out.reshape(_ATTRS, B, n_anch * HW).transpose(1, 2, 0)
